# Initial kernel scaffold; baseline (speedup 1.0000x reference)
import jax
import jax.numpy as jnp
from jax.experimental import pallas as pl


def kernel(ids, Ps, pos, top_k, W):
    raise NotImplementedError("write your pallas kernel here")


# speedup vs baseline: 1.6323x; 1.6323x over previous
import functools

import jax
import jax.numpy as jnp
from jax import lax
from jax.experimental import pallas as pl
from jax.experimental.pallas import tpu as pltpu
from jax.experimental.pallas import tpu_sc as plsc

_LANES = 16


def _transpose_norm_kernel(nv, bv, w_ref, wt_ref, nrm_ref):
    j = pl.program_id(0)
    nblk = pl.num_programs(0)
    w = w_ref[...]
    col = j * bv + lax.broadcasted_iota(jnp.int32, w.shape, 1)
    wm = jnp.where(col < nv, w, 0.0)
    wt_ref[...] = wm.T
    part = jnp.sum(wm * wm, axis=1, keepdims=True)

    @pl.when(j == 0)
    def _():
        nrm_ref[...] = part

    @pl.when(j > 0)
    def _():
        nrm_ref[...] += part

    @pl.when(j == nblk - 1)
    def _():
        nrm_ref[...] = jnp.sqrt(nrm_ref[...])


def _transpose_and_norms(W, bv=1024, interpret=False):
    k, nv = W.shape
    grid = pl.cdiv(nv, bv)
    wt, nrm = pl.pallas_call(
        functools.partial(_transpose_norm_kernel, nv, bv),
        grid=(grid,),
        in_specs=[pl.BlockSpec((k, bv), lambda j: (0, j))],
        out_specs=[
            pl.BlockSpec((bv, k), lambda j: (j, 0)),
            pl.BlockSpec((k, 1), lambda j: (0, 0)),
        ],
        out_shape=[
            jax.ShapeDtypeStruct((nv, k), jnp.float32),
            jax.ShapeDtypeStruct((k, 1), jnp.float32),
        ],
        interpret=interpret,
    )(W)
    return wt, nrm.reshape(k)


def _make_sc_kernel(batch, win, k, pw, gw):
    info = plsc.get_sparse_core_info()
    nc, ns = info.num_cores, info.num_subcores
    nw = nc * ns
    bpw = batch // nw
    gpw = bpw // 2
    mesh = plsc.VectorSubcoreMesh(core_axis_name="c", subcore_axis_name="s")
    nchunk = k // _LANES

    @functools.partial(
        pl.kernel,
        out_type=jax.ShapeDtypeStruct((nw, _LANES), jnp.float32),
        mesh=mesh,
        compiler_params=pltpu.CompilerParams(needs_layout_passes=False),
        scratch_types=[
            pltpu.VMEM((gpw, gw), jnp.int32),
            pltpu.VMEM((bpw, pw), jnp.float32),
            pltpu.VMEM((k,), jnp.float32),
            pltpu.VMEM((gw, k), jnp.float32),
            pltpu.VMEM((gw, k), jnp.float32),
            pltpu.VMEM((_LANES,), jnp.float32),
            pltpu.SemaphoreType.DMA,
            pltpu.SemaphoreType.DMA,
        ],
    )
    def sc_kernel(wt_hbm, idg_hbm, ps_hbm, nrm_hbm, out_hbm,
                  idg_v, ps_v, nrm_v, buf_a, buf_b, cacc_v, sem_a, sem_b):
        wid = lax.axis_index("s") * nc + lax.axis_index("c")
        pltpu.sync_copy(idg_hbm.at[pl.ds(wid * gpw, gpw)], idg_v)
        pltpu.sync_copy(ps_hbm.at[pl.ds(wid * bpw, bpw)], ps_v)
        pltpu.sync_copy(nrm_hbm, nrm_v)

        lanes = lax.iota(jnp.int32, _LANES)

        def one_batch(buf, rbase, b, cacc):
            def chunk_body(j, carry):
                bv, bi = carry
                off = j * _LANES
                s = buf[rbase, pl.ds(off, _LANES)]
                for w in range(1, win):
                    s = s + buf[rbase + w, pl.ds(off, _LANES)]
                m = jnp.max(s)
                li = jnp.min(jnp.where(s == m, lanes, _LANES))
                cand = off + li
                upd = m > bv
                return jnp.where(upd, m, bv), jnp.where(upd, cand, bi)

            _, mu = lax.fori_loop(
                0, nchunk, chunk_body,
                (jnp.float32(-3.0e38), jnp.int32(0)))

            mu_vec = jnp.full((_LANES,), mu, jnp.int32)
            w1 = rbase + lanes
            w2 = rbase + jnp.minimum(lanes + _LANES, win - 1)
            v1 = plsc.load_gather(buf, [w1, mu_vec])
            v2 = plsc.load_gather(buf, [w2, mu_vec])
            p1 = ps_v[b, pl.ds(0, _LANES)]
            p2 = ps_v[b, pl.ds(_LANES, _LANES)]
            num = jnp.sum(v1 * p1 + v2 * p2)
            den = plsc.load_gather(nrm_v, [mu_vec])
            c = num / den
            return cacc + jnp.where(lanes == 0, c, 0.0)

        def start_gather(gi, buf, sem):
            return pltpu.async_copy(wt_hbm.at[idg_v.at[gi]], buf, sem)

        start_gather(jnp.int32(0), buf_a, sem_a)

        def iter_body(g, cacc):
            start_gather(2 * g + 1, buf_b, sem_b)
            pltpu.make_async_copy(wt_hbm.at[idg_v.at[2 * g]], buf_a,
                                  sem_a).wait()
            cacc = one_batch(buf_a, 0, 4 * g, cacc)
            cacc = one_batch(buf_a, win, 4 * g + 1, cacc)
            start_gather(jnp.minimum(2 * g + 2, gpw - 1), buf_a, sem_a)
            pltpu.make_async_copy(wt_hbm.at[idg_v.at[2 * g + 1]], buf_b,
                                  sem_b).wait()
            cacc = one_batch(buf_b, 0, 4 * g + 2, cacc)
            cacc = one_batch(buf_b, win, 4 * g + 3, cacc)
            return cacc

        cacc = lax.fori_loop(0, gpw // 2, iter_body,
                             jnp.zeros((_LANES,), jnp.float32))
        pltpu.make_async_copy(wt_hbm.at[idg_v.at[gpw - 1]], buf_a,
                              sem_a).wait()
        cacc_v[...] = cacc
        pltpu.sync_copy(cacc_v, out_hbm.at[wid])

    return sc_kernel


def kernel(ids, Ps, pos, top_k, W):
    del pos, top_k
    k, nv = W.shape
    batch, win = Ps.shape
    pw = 2 * _LANES
    gw = ((2 * win + 15) // 16) * 16

    wt, nrm = _transpose_and_norms(W)
    ids2 = ids.reshape(batch, win)
    idg = ids2.reshape(batch // 2, 2 * win)
    idg = jnp.concatenate(
        [idg, jnp.broadcast_to(idg[:, -1:], (batch // 2, gw - 2 * win))],
        axis=1)
    ps_pad = jnp.zeros((batch, pw), jnp.float32).at[:, :win].set(Ps)

    partials = _make_sc_kernel(batch, win, k, pw, gw)(wt, idg, ps_pad, nrm)
    return -jnp.sum(partials)
